# baseline (speedup 1.0000x reference)
import functools

import jax
import jax.numpy as jnp
from jax import lax
from jax.experimental import pallas as pl
from jax.experimental.pallas import tpu as pltpu
from jax.experimental.pallas import tpu_sc as plsc

N = 10000
E = 320000
B = 128
D = 128

NC = 2
NS = 16
NW = NC * NS
EPW = E // NW
CH = 64
NFULL = EPW // CH
TAIL = EPW - NFULL * CH
NBUF = 3
RPW = N // NS

NB = 10
RB = N // NB


def _mesh():
    return plsc.VectorSubcoreMesh(core_axis_name="c", subcore_axis_name="s",
                                  num_cores=NC, num_subcores=NS)


def _sc_params():
    return pltpu.CompilerParams(needs_layout_passes=False,
                                use_tc_tiling_on_sc=False)


@functools.cache
def _deg_call():
    @functools.partial(
        pl.kernel,
        out_type=jax.ShapeDtypeStruct((NW, N), jnp.float32),
        mesh=_mesh(),
        compiler_params=_sc_params(),
        scratch_types=[
            pltpu.VMEM((N,), jnp.float32),
            pltpu.VMEM((EPW,), jnp.int32),
            pltpu.VMEM((EPW,), jnp.float32),
        ],
    )
    def deg_kernel(col_hbm, ew_hbm, zn_hbm, out_hbm, deg_v, col_v, ew_v):
        wid = lax.axis_index("s") * NC + lax.axis_index("c")
        pltpu.sync_copy(zn_hbm, deg_v)
        pltpu.sync_copy(col_hbm.at[wid], col_v)
        pltpu.sync_copy(ew_hbm.at[wid], ew_v)

        def body(i, carry):
            idx = col_v[pl.ds(i * 16, 16)]
            w = ew_v[pl.ds(i * 16, 16)]
            plsc.addupdate_scatter(deg_v, [idx], w)
            return carry

        lax.fori_loop(0, EPW // 16, body, 0, unroll=4)
        pltpu.sync_copy(deg_v, out_hbm.at[wid])

    return deg_kernel


@functools.cache
def _msg_call():
    @functools.partial(
        pl.kernel,
        out_type=jax.ShapeDtypeStruct((NC * N, D), jnp.float32),
        mesh=_mesh(),
        compiler_params=_sc_params(),
        scratch_types=[
            pltpu.VMEM_SHARED((N, D), jnp.float32),
            pltpu.VMEM((NFULL, CH), jnp.int32),
            pltpu.VMEM((NFULL, CH), jnp.int32),
            pltpu.VMEM((TAIL,), jnp.int32),
            pltpu.VMEM((TAIL,), jnp.int32),
            [pltpu.VMEM((CH,), jnp.float32)] * NBUF,
            [pltpu.VMEM((CH, D), jnp.float32)] * NBUF,
            [pltpu.SemaphoreType.DMA] * NBUF,
            [pltpu.SemaphoreType.DMA] * NBUF,
        ],
    )
    def msg_kernel(y_hbm, rowm_hbm, colm_hbm, rowt_hbm, colt_hbm, ew_hbm,
                   z2_hbm, out_hbm,
                   acc_s, rowm_v, colm_v, rowt_v, colt_v, ewc, rows, gsem,
                   ssem):
        cid = lax.axis_index("c")
        sid = lax.axis_index("s")
        wid = sid * NC + cid

        pltpu.sync_copy(z2_hbm.at[pl.ds(sid * RPW, RPW)],
                        acc_s.at[pl.ds(sid * RPW, RPW)])
        pltpu.sync_copy(rowm_hbm.at[wid], rowm_v)
        pltpu.sync_copy(colm_hbm.at[wid], colm_v)
        pltpu.sync_copy(rowt_hbm.at[wid], rowt_v)
        pltpu.sync_copy(colt_hbm.at[wid], colt_v)
        plsc.subcore_barrier()

        def scale_rows(ref, ewc, nrows):
            def srow(i, carry):
                eb = plsc.load_gather(ewc, [jnp.full((16,), i, jnp.int32)])
                for g in range(D // 16):
                    sl = (i, pl.ds(g * 16, 16))
                    ref[sl] = ref[sl] * eb
                return carry
            lax.fori_loop(0, nrows, srow, 0, unroll=8)

        def issue(j, b):
            pltpu.async_copy(y_hbm.at[rowm_v.at[j]], rows[b], gsem[b])
            pltpu.async_copy(ew_hbm.at[wid, pl.ds(j * CH, CH)], ewc[b],
                             gsem[b])

        def drain(b):
            pltpu.make_async_copy(y_hbm.at[pl.ds(0, CH)], rows[b],
                                  gsem[b]).wait()
            pltpu.make_async_copy(ew_hbm.at[0, pl.ds(0, CH)], ewc[b],
                                  gsem[b]).wait()

        def wait_scatter(b):
            pltpu.make_async_copy(y_hbm.at[pl.ds(0, CH)], rows[b],
                                  ssem[b]).wait()

        issue(0, 0)
        issue(1, 1)

        def body(t, carry):
            j0 = NBUF * t
            for b in range(NBUF):
                j = j0 + b
                drain(b)
                scale_rows(rows[b], ewc[b], CH)
                pltpu.async_copy(rows[b], acc_s.at[colm_v.at[j]], ssem[b],
                                 add=True)
                bn = (b + 2) % NBUF
                @pl.when(j >= 1)
                def _():
                    wait_scatter(bn)

                @pl.when(j + 2 < NFULL)
                def _():
                    issue(j + 2, bn)
            return carry

        lax.fori_loop(0, NFULL // NBUF, body, 0)
        wait_scatter((NFULL - 1) % NBUF)

        pltpu.async_copy(y_hbm.at[rowt_v], rows[0].at[pl.ds(0, TAIL)],
                         gsem[0]).wait()
        pltpu.sync_copy(ew_hbm.at[wid, pl.ds(NFULL * CH, TAIL)],
                        ewc[0].at[pl.ds(0, TAIL)])
        scale_rows(rows[0], ewc[0], TAIL)
        pltpu.sync_copy(rows[0].at[pl.ds(0, TAIL)], acc_s.at[colt_v],
                        add=True)

        plsc.subcore_barrier()
        pltpu.sync_copy(acc_s.at[pl.ds(sid * RPW, RPW)],
                        out_hbm.at[pl.ds(cid * N + sid * RPW, RPW)])

    return msg_kernel


def _tc1_body(degT_ref, x_ref, w_ref, y_ref, dinv_ref):
    deg = 1.0 + jnp.sum(degT_ref[...], axis=1, keepdims=True)
    dinv = lax.rsqrt(deg)
    xw = jnp.dot(x_ref[...], w_ref[...], preferred_element_type=jnp.float32)
    y_ref[...] = xw * dinv
    dinv_ref[...] = dinv


@functools.cache
def _tc1_call():
    return pl.pallas_call(
        _tc1_body,
        grid=(NB,),
        in_specs=[
            pl.BlockSpec((RB, NW), lambda i: (i, 0)),
            pl.BlockSpec((RB, D), lambda i: (i, 0)),
            pl.BlockSpec((D, D), lambda i: (0, 0)),
        ],
        out_specs=[
            pl.BlockSpec((RB, D), lambda i: (i, 0)),
            pl.BlockSpec((RB, 1), lambda i: (i, 0)),
        ],
        out_shape=[
            jax.ShapeDtypeStruct((N, D), jnp.float32),
            jax.ShapeDtypeStruct((N, 1), jnp.float32),
        ],
    )


def _tc2_body(pa_ref, pb_ref, y1_ref, dinv_ref, b1_ref, w2_ref, y2_ref):
    dinv = dinv_ref[...]
    s = pa_ref[...] + pb_ref[...] + y1_ref[...]
    h = jnp.maximum(dinv * s + b1_ref[...], 0.0)
    y2_ref[...] = jnp.dot(h, w2_ref[...],
                          preferred_element_type=jnp.float32) * dinv


@functools.cache
def _tc2_call():
    return pl.pallas_call(
        _tc2_body,
        grid=(NB,),
        in_specs=[
            pl.BlockSpec((RB, D), lambda i: (i, 0)),
            pl.BlockSpec((RB, D), lambda i: (i + NB, 0)),
            pl.BlockSpec((RB, D), lambda i: (i, 0)),
            pl.BlockSpec((RB, 1), lambda i: (i, 0)),
            pl.BlockSpec((1, D), lambda i: (0, 0)),
            pl.BlockSpec((D, D), lambda i: (0, 0)),
        ],
        out_specs=pl.BlockSpec((RB, D), lambda i: (i, 0)),
        out_shape=jax.ShapeDtypeStruct((N, D), jnp.float32),
    )


def _tc3_body(pa_ref, pb_ref, y2_ref, dinv_ref, b2_ref, bt_ref,
              out_ref, cnt_ref):
    i = pl.program_id(0)
    s = pa_ref[...] + pb_ref[...] + y2_ref[...]
    z = jnp.maximum(dinv_ref[...] * s + b2_ref[...], 0.0)
    bt = bt_ref[0]
    oh = (bt == lax.broadcasted_iota(jnp.int32, (1, B), 1)
          ).astype(jnp.float32)
    blk_sums = lax.dot_general(oh, z, (((0,), (0,)), ((), ())),
                               preferred_element_type=jnp.float32)
    blk_cnt = lax.dot_general(oh, jnp.ones((RB, 1), jnp.float32),
                              (((0,), (0,)), ((), ())),
                              preferred_element_type=jnp.float32)

    @pl.when(i == 0)
    def _():
        out_ref[...] = blk_sums
        cnt_ref[...] = blk_cnt

    @pl.when(i > 0)
    def _():
        out_ref[...] = out_ref[...] + blk_sums
        cnt_ref[...] = cnt_ref[...] + blk_cnt

    @pl.when(i == NB - 1)
    def _():
        out_ref[...] = out_ref[...] / jnp.maximum(cnt_ref[...], 1.0)


@functools.cache
def _tc3_call():
    return pl.pallas_call(
        _tc3_body,
        grid=(NB,),
        in_specs=[
            pl.BlockSpec((RB, D), lambda i: (i, 0)),
            pl.BlockSpec((RB, D), lambda i: (i + NB, 0)),
            pl.BlockSpec((RB, D), lambda i: (i, 0)),
            pl.BlockSpec((RB, 1), lambda i: (i, 0)),
            pl.BlockSpec((1, D), lambda i: (0, 0)),
            pl.BlockSpec((1, RB, 1), lambda i: (i, 0, 0)),
        ],
        out_specs=[
            pl.BlockSpec((B, B), lambda i: (0, 0)),
            pl.BlockSpec((B, 1), lambda i: (0, 0)),
        ],
        out_shape=[
            jax.ShapeDtypeStruct((B, B), jnp.float32),
            jax.ShapeDtypeStruct((B, 1), jnp.float32),
        ],
    )


def kernel(x, edge_index, edge_weight, batch, W1, b1, W2, b2):
    row = edge_index[0].astype(jnp.int32)
    col = edge_index[1].astype(jnp.int32)
    ew = edge_weight

    col2 = col.reshape(NW, EPW)
    ew2 = ew.reshape(NW, EPW)
    row2 = row.reshape(NW, EPW)
    rowm = row2[:, :NFULL * CH].reshape(NW, NFULL, CH)
    rowt = row2[:, NFULL * CH:]
    colm = col2[:, :NFULL * CH].reshape(NW, NFULL, CH)
    colt = col2[:, NFULL * CH:]

    zn = jnp.zeros((N,), jnp.float32)
    z2 = jnp.zeros((N, D), jnp.float32)

    deg_parts = _deg_call()(col2, ew2, zn)
    degT = deg_parts.T
    y1, dinv = _tc1_call()(degT, x, W1)

    parts1 = _msg_call()(y1, rowm, colm, rowt, colt, ew2, z2)
    y2 = _tc2_call()(parts1, parts1, y1, dinv, b1.reshape(1, D), W2)

    parts2 = _msg_call()(y2, rowm, colm, rowt, colt, ew2, z2)
    mean, _ = _tc3_call()(parts2, parts2, y2, dinv, b2.reshape(1, D),
                          batch.astype(jnp.int32).reshape(NB, RB, 1))
    return mean


# speedup vs baseline: 1.0053x; 1.0053x over previous
import functools

import jax
import jax.numpy as jnp
from jax import lax
from jax.experimental import pallas as pl
from jax.experimental.pallas import tpu as pltpu
from jax.experimental.pallas import tpu_sc as plsc

N = 10000
E = 320000
B = 128
D = 128

NC = 2
NS = 16
NW = NC * NS
EPW = E // NW
CH = 64
NFULL = EPW // CH
TAIL = EPW - NFULL * CH
NBUF = 3
RPW = N // NS

NB = 10
RB = N // NB


def _mesh():
    return plsc.VectorSubcoreMesh(core_axis_name="c", subcore_axis_name="s",
                                  num_cores=NC, num_subcores=NS)


def _sc_params():
    return pltpu.CompilerParams(needs_layout_passes=False,
                                use_tc_tiling_on_sc=False)


@functools.cache
def _deg_call():
    @functools.partial(
        pl.kernel,
        out_type=jax.ShapeDtypeStruct((NW, N), jnp.float32),
        mesh=_mesh(),
        compiler_params=_sc_params(),
        scratch_types=[
            pltpu.VMEM((N,), jnp.float32),
            pltpu.VMEM((EPW,), jnp.int32),
            pltpu.VMEM((EPW,), jnp.float32),
        ],
    )
    def deg_kernel(col_hbm, ew_hbm, zn_hbm, out_hbm, deg_v, col_v, ew_v):
        wid = lax.axis_index("s") * NC + lax.axis_index("c")
        pltpu.sync_copy(zn_hbm, deg_v)
        pltpu.sync_copy(col_hbm.at[wid], col_v)
        pltpu.sync_copy(ew_hbm.at[wid], ew_v)

        def body(i, carry):
            idx = col_v[pl.ds(i * 16, 16)]
            w = ew_v[pl.ds(i * 16, 16)]
            plsc.addupdate_scatter(deg_v, [idx], w)
            return carry

        lax.fori_loop(0, EPW // 16, body, 0, unroll=4)
        pltpu.sync_copy(deg_v, out_hbm.at[wid])

    return deg_kernel


@functools.cache
def _msg_call():
    @functools.partial(
        pl.kernel,
        out_type=jax.ShapeDtypeStruct((NC * N, D), jnp.float32),
        mesh=_mesh(),
        compiler_params=_sc_params(),
        scratch_types=[
            pltpu.VMEM_SHARED((N, D), jnp.float32),
            pltpu.VMEM((NFULL, CH), jnp.int32),
            pltpu.VMEM((NFULL, CH), jnp.int32),
            pltpu.VMEM((TAIL,), jnp.int32),
            pltpu.VMEM((TAIL,), jnp.int32),
            [pltpu.VMEM((CH,), jnp.float32)] * NBUF,
            [pltpu.VMEM((CH, D), jnp.float32)] * NBUF,
            [pltpu.SemaphoreType.DMA] * NBUF,
            [pltpu.SemaphoreType.DMA] * NBUF,
        ],
    )
    def msg_kernel(y_hbm, rowm_hbm, colm_hbm, rowt_hbm, colt_hbm, ew_hbm,
                   z2_hbm, out_hbm,
                   acc_s, rowm_v, colm_v, rowt_v, colt_v, ewc, rows, gsem,
                   ssem):
        cid = lax.axis_index("c")
        sid = lax.axis_index("s")
        wid = sid * NC + cid

        pltpu.sync_copy(rowm_hbm.at[wid], rowm_v)
        pltpu.sync_copy(colm_hbm.at[wid], colm_v)

        def scale_rows(ref, ewc, nrows):
            def srow(i, carry):
                eb = plsc.load_gather(ewc, [jnp.full((16,), i, jnp.int32)])
                for g in range(D // 16):
                    sl = (i, pl.ds(g * 16, 16))
                    ref[sl] = ref[sl] * eb
                return carry
            lax.fori_loop(0, nrows, srow, 0, unroll=8)

        def issue(j, b):
            pltpu.async_copy(y_hbm.at[rowm_v.at[j]], rows[b], gsem[b])
            pltpu.async_copy(ew_hbm.at[wid, pl.ds(j * CH, CH)], ewc[b],
                             gsem[b])

        def drain(b):
            pltpu.make_async_copy(y_hbm.at[pl.ds(0, CH)], rows[b],
                                  gsem[b]).wait()
            pltpu.make_async_copy(ew_hbm.at[0, pl.ds(0, CH)], ewc[b],
                                  gsem[b]).wait()

        def wait_scatter(b):
            pltpu.make_async_copy(y_hbm.at[pl.ds(0, CH)], rows[b],
                                  ssem[b]).wait()

        issue(0, 0)
        issue(1, 1)

        pltpu.sync_copy(z2_hbm.at[pl.ds(sid * RPW, RPW)],
                        acc_s.at[pl.ds(sid * RPW, RPW)])
        pltpu.sync_copy(rowt_hbm.at[wid], rowt_v)
        pltpu.sync_copy(colt_hbm.at[wid], colt_v)
        plsc.subcore_barrier()

        def body(t, carry):
            j0 = NBUF * t
            for b in range(NBUF):
                j = j0 + b
                drain(b)
                scale_rows(rows[b], ewc[b], CH)
                pltpu.async_copy(rows[b], acc_s.at[colm_v.at[j]], ssem[b],
                                 add=True)
                bn = (b + 2) % NBUF
                @pl.when(j >= 1)
                def _():
                    wait_scatter(bn)

                @pl.when(j + 2 < NFULL)
                def _():
                    issue(j + 2, bn)
            return carry

        lax.fori_loop(0, NFULL // NBUF, body, 0)
        wait_scatter((NFULL - 1) % NBUF)

        pltpu.async_copy(y_hbm.at[rowt_v], rows[0].at[pl.ds(0, TAIL)],
                         gsem[0]).wait()
        pltpu.sync_copy(ew_hbm.at[wid, pl.ds(NFULL * CH, TAIL)],
                        ewc[0].at[pl.ds(0, TAIL)])
        scale_rows(rows[0], ewc[0], TAIL)
        pltpu.sync_copy(rows[0].at[pl.ds(0, TAIL)], acc_s.at[colt_v],
                        add=True)

        plsc.subcore_barrier()
        pltpu.sync_copy(acc_s.at[pl.ds(sid * RPW, RPW)],
                        out_hbm.at[pl.ds(cid * N + sid * RPW, RPW)])

    return msg_kernel


def _tc1_body(degT_ref, x_ref, w_ref, y_ref, dinv_ref):
    deg = 1.0 + jnp.sum(degT_ref[...], axis=1, keepdims=True)
    dinv = lax.rsqrt(deg)
    xw = jnp.dot(x_ref[...], w_ref[...], preferred_element_type=jnp.float32)
    y_ref[...] = xw * dinv
    dinv_ref[...] = dinv


@functools.cache
def _tc1_call():
    return pl.pallas_call(
        _tc1_body,
        grid=(NB,),
        in_specs=[
            pl.BlockSpec((RB, NW), lambda i: (i, 0)),
            pl.BlockSpec((RB, D), lambda i: (i, 0)),
            pl.BlockSpec((D, D), lambda i: (0, 0)),
        ],
        out_specs=[
            pl.BlockSpec((RB, D), lambda i: (i, 0)),
            pl.BlockSpec((RB, 1), lambda i: (i, 0)),
        ],
        out_shape=[
            jax.ShapeDtypeStruct((N, D), jnp.float32),
            jax.ShapeDtypeStruct((N, 1), jnp.float32),
        ],
    )


def _tc2_body(pa_ref, pb_ref, y1_ref, dinv_ref, b1_ref, w2_ref, y2_ref):
    dinv = dinv_ref[...]
    s = pa_ref[...] + pb_ref[...] + y1_ref[...]
    h = jnp.maximum(dinv * s + b1_ref[...], 0.0)
    y2_ref[...] = jnp.dot(h, w2_ref[...],
                          preferred_element_type=jnp.float32) * dinv


@functools.cache
def _tc2_call():
    return pl.pallas_call(
        _tc2_body,
        grid=(NB,),
        in_specs=[
            pl.BlockSpec((RB, D), lambda i: (i, 0)),
            pl.BlockSpec((RB, D), lambda i: (i + NB, 0)),
            pl.BlockSpec((RB, D), lambda i: (i, 0)),
            pl.BlockSpec((RB, 1), lambda i: (i, 0)),
            pl.BlockSpec((1, D), lambda i: (0, 0)),
            pl.BlockSpec((D, D), lambda i: (0, 0)),
        ],
        out_specs=pl.BlockSpec((RB, D), lambda i: (i, 0)),
        out_shape=jax.ShapeDtypeStruct((N, D), jnp.float32),
    )


def _tc3_body(pa_ref, pb_ref, y2_ref, dinv_ref, b2_ref, bt_ref,
              out_ref, cnt_ref):
    i = pl.program_id(0)
    s = pa_ref[...] + pb_ref[...] + y2_ref[...]
    z = jnp.maximum(dinv_ref[...] * s + b2_ref[...], 0.0)
    bt = bt_ref[0]
    oh = (bt == lax.broadcasted_iota(jnp.int32, (1, B), 1)
          ).astype(jnp.float32)
    blk_sums = lax.dot_general(oh, z, (((0,), (0,)), ((), ())),
                               preferred_element_type=jnp.float32)
    blk_cnt = lax.dot_general(oh, jnp.ones((RB, 1), jnp.float32),
                              (((0,), (0,)), ((), ())),
                              preferred_element_type=jnp.float32)

    @pl.when(i == 0)
    def _():
        out_ref[...] = blk_sums
        cnt_ref[...] = blk_cnt

    @pl.when(i > 0)
    def _():
        out_ref[...] = out_ref[...] + blk_sums
        cnt_ref[...] = cnt_ref[...] + blk_cnt

    @pl.when(i == NB - 1)
    def _():
        out_ref[...] = out_ref[...] / jnp.maximum(cnt_ref[...], 1.0)


@functools.cache
def _tc3_call():
    return pl.pallas_call(
        _tc3_body,
        grid=(NB,),
        in_specs=[
            pl.BlockSpec((RB, D), lambda i: (i, 0)),
            pl.BlockSpec((RB, D), lambda i: (i + NB, 0)),
            pl.BlockSpec((RB, D), lambda i: (i, 0)),
            pl.BlockSpec((RB, 1), lambda i: (i, 0)),
            pl.BlockSpec((1, D), lambda i: (0, 0)),
            pl.BlockSpec((1, RB, 1), lambda i: (i, 0, 0)),
        ],
        out_specs=[
            pl.BlockSpec((B, B), lambda i: (0, 0)),
            pl.BlockSpec((B, 1), lambda i: (0, 0)),
        ],
        out_shape=[
            jax.ShapeDtypeStruct((B, B), jnp.float32),
            jax.ShapeDtypeStruct((B, 1), jnp.float32),
        ],
    )


def kernel(x, edge_index, edge_weight, batch, W1, b1, W2, b2):
    row = edge_index[0].astype(jnp.int32)
    col = edge_index[1].astype(jnp.int32)
    ew = edge_weight

    col2 = col.reshape(NW, EPW)
    ew2 = ew.reshape(NW, EPW)
    row2 = row.reshape(NW, EPW)
    rowm = row2[:, :NFULL * CH].reshape(NW, NFULL, CH)
    rowt = row2[:, NFULL * CH:]
    colm = col2[:, :NFULL * CH].reshape(NW, NFULL, CH)
    colt = col2[:, NFULL * CH:]

    zn = jnp.zeros((N,), jnp.float32)
    z2 = jnp.zeros((N, D), jnp.float32)

    deg_parts = _deg_call()(col2, ew2, zn)
    degT = deg_parts.T
    y1, dinv = _tc1_call()(degT, x, W1)

    parts1 = _msg_call()(y1, rowm, colm, rowt, colt, ew2, z2)
    y2 = _tc2_call()(parts1, parts1, y1, dinv, b1.reshape(1, D), W2)

    parts2 = _msg_call()(y2, rowm, colm, rowt, colt, ew2, z2)
    mean, _ = _tc3_call()(parts2, parts2, y2, dinv, b2.reshape(1, D),
                          batch.astype(jnp.int32).reshape(NB, RB, 1))
    return mean


# speedup vs baseline: 1.1536x; 1.1475x over previous
import functools

import jax
import jax.numpy as jnp
from jax import lax
from jax.experimental import pallas as pl
from jax.experimental.pallas import tpu as pltpu
from jax.experimental.pallas import tpu_sc as plsc

N = 10000
E = 320000
B = 128
D = 128

NC = 2
NS = 16
NW = NC * NS
EPW = E // NW
CH = 48
NFULL = EPW // CH
TAIL = EPW - NFULL * CH
NBUF = 4
RPW = N // NS

NB = 10
RB = N // NB


def _mesh():
    return plsc.VectorSubcoreMesh(core_axis_name="c", subcore_axis_name="s",
                                  num_cores=NC, num_subcores=NS)


def _sc_params():
    return pltpu.CompilerParams(needs_layout_passes=False,
                                use_tc_tiling_on_sc=False)


@functools.cache
def _deg_call():
    @functools.partial(
        pl.kernel,
        out_type=jax.ShapeDtypeStruct((NW, N), jnp.float32),
        mesh=_mesh(),
        compiler_params=_sc_params(),
        scratch_types=[
            pltpu.VMEM((N,), jnp.float32),
            pltpu.VMEM((EPW,), jnp.int32),
            pltpu.VMEM((EPW,), jnp.float32),
        ],
    )
    def deg_kernel(col_hbm, ew_hbm, zn_hbm, out_hbm, deg_v, col_v, ew_v):
        wid = lax.axis_index("s") * NC + lax.axis_index("c")
        pltpu.sync_copy(zn_hbm, deg_v)
        pltpu.sync_copy(col_hbm.at[wid], col_v)
        pltpu.sync_copy(ew_hbm.at[wid], ew_v)

        def body(i, carry):
            idx = col_v[pl.ds(i * 16, 16)]
            w = ew_v[pl.ds(i * 16, 16)]
            plsc.addupdate_scatter(deg_v, [idx], w)
            return carry

        lax.fori_loop(0, EPW // 16, body, 0, unroll=4)
        pltpu.sync_copy(deg_v, out_hbm.at[wid])

    return deg_kernel


@functools.cache
def _msg_call():
    @functools.partial(
        pl.kernel,
        out_type=jax.ShapeDtypeStruct((NC * N, D), jnp.float32),
        mesh=_mesh(),
        compiler_params=_sc_params(),
        scratch_types=[
            pltpu.VMEM_SHARED((N, D), jnp.float32),
            pltpu.VMEM((NFULL, CH), jnp.int32),
            pltpu.VMEM((NFULL, CH), jnp.int32),
            pltpu.VMEM((TAIL,), jnp.int32),
            pltpu.VMEM((TAIL,), jnp.int32),
            [pltpu.VMEM((CH,), jnp.float32)] * NBUF,
            [pltpu.VMEM((CH, D), jnp.float32)] * NBUF,
            [pltpu.SemaphoreType.DMA] * NBUF,
            [pltpu.SemaphoreType.DMA] * NBUF,
        ],
    )
    def msg_kernel(y_hbm, rowm_hbm, colm_hbm, rowt_hbm, colt_hbm, ew_hbm,
                   z2_hbm, out_hbm,
                   acc_s, rowm_v, colm_v, rowt_v, colt_v, ewc, rows, gsem,
                   ssem):
        cid = lax.axis_index("c")
        sid = lax.axis_index("s")
        wid = sid * NC + cid

        pltpu.sync_copy(rowm_hbm.at[wid], rowm_v)
        pltpu.sync_copy(colm_hbm.at[wid], colm_v)

        def scale_rows(ref, ewc, nrows):
            def srow(i, carry):
                eb = plsc.load_gather(ewc, [jnp.full((16,), i, jnp.int32)])
                for g in range(D // 16):
                    sl = (i, pl.ds(g * 16, 16))
                    ref[sl] = ref[sl] * eb
                return carry
            lax.fori_loop(0, nrows, srow, 0, unroll=8)

        def issue(j, b):
            pltpu.async_copy(y_hbm.at[rowm_v.at[j]], rows[b], gsem[b])
            pltpu.async_copy(ew_hbm.at[wid, pl.ds(j * CH, CH)], ewc[b],
                             gsem[b])

        def drain(b):
            pltpu.make_async_copy(y_hbm.at[pl.ds(0, CH)], rows[b],
                                  gsem[b]).wait()
            pltpu.make_async_copy(ew_hbm.at[0, pl.ds(0, CH)], ewc[b],
                                  gsem[b]).wait()

        def wait_scatter(b):
            pltpu.make_async_copy(y_hbm.at[pl.ds(0, CH)], rows[b],
                                  ssem[b]).wait()

        for jb in range(NBUF - 1):
            issue(jb, jb)

        pltpu.sync_copy(z2_hbm.at[pl.ds(sid * RPW, RPW)],
                        acc_s.at[pl.ds(sid * RPW, RPW)])
        pltpu.sync_copy(rowt_hbm.at[wid], rowt_v)
        pltpu.sync_copy(colt_hbm.at[wid], colt_v)
        plsc.subcore_barrier()

        def body(t, carry):
            j0 = NBUF * t
            for b in range(NBUF):
                j = j0 + b
                drain(b)
                scale_rows(rows[b], ewc[b], CH)
                pltpu.async_copy(rows[b], acc_s.at[colm_v.at[j]], ssem[b],
                                 add=True)
                bn = (b + NBUF - 1) % NBUF
                @pl.when(j >= 1)
                def _():
                    wait_scatter(bn)

                @pl.when(j + NBUF - 1 < NFULL)
                def _():
                    issue(j + NBUF - 1, bn)
            return carry

        lax.fori_loop(0, NFULL // NBUF, body, 0)
        wait_scatter((NFULL - 1) % NBUF)

        pltpu.async_copy(y_hbm.at[rowt_v], rows[0].at[pl.ds(0, TAIL)],
                         gsem[0]).wait()
        pltpu.sync_copy(ew_hbm.at[wid, pl.ds(NFULL * CH, TAIL)],
                        ewc[0].at[pl.ds(0, TAIL)])
        scale_rows(rows[0], ewc[0], TAIL)
        pltpu.sync_copy(rows[0].at[pl.ds(0, TAIL)], acc_s.at[colt_v],
                        add=True)

        plsc.subcore_barrier()
        pltpu.sync_copy(acc_s.at[pl.ds(sid * RPW, RPW)],
                        out_hbm.at[pl.ds(cid * N + sid * RPW, RPW)])

    return msg_kernel


def _tc1_body(degT_ref, x_ref, w_ref, y_ref, dinv_ref):
    deg = 1.0 + jnp.sum(degT_ref[...], axis=1, keepdims=True)
    dinv = lax.rsqrt(deg)
    xw = jnp.dot(x_ref[...], w_ref[...], preferred_element_type=jnp.float32)
    y_ref[...] = xw * dinv
    dinv_ref[...] = dinv


@functools.cache
def _tc1_call():
    return pl.pallas_call(
        _tc1_body,
        grid=(NB,),
        in_specs=[
            pl.BlockSpec((RB, NW), lambda i: (i, 0)),
            pl.BlockSpec((RB, D), lambda i: (i, 0)),
            pl.BlockSpec((D, D), lambda i: (0, 0)),
        ],
        out_specs=[
            pl.BlockSpec((RB, D), lambda i: (i, 0)),
            pl.BlockSpec((RB, 1), lambda i: (i, 0)),
        ],
        out_shape=[
            jax.ShapeDtypeStruct((N, D), jnp.float32),
            jax.ShapeDtypeStruct((N, 1), jnp.float32),
        ],
    )


def _tc2_body(pa_ref, pb_ref, y1_ref, dinv_ref, b1_ref, w2_ref, y2_ref):
    dinv = dinv_ref[...]
    s = pa_ref[...] + pb_ref[...] + y1_ref[...]
    h = jnp.maximum(dinv * s + b1_ref[...], 0.0)
    y2_ref[...] = jnp.dot(h, w2_ref[...],
                          preferred_element_type=jnp.float32) * dinv


@functools.cache
def _tc2_call():
    return pl.pallas_call(
        _tc2_body,
        grid=(NB,),
        in_specs=[
            pl.BlockSpec((RB, D), lambda i: (i, 0)),
            pl.BlockSpec((RB, D), lambda i: (i + NB, 0)),
            pl.BlockSpec((RB, D), lambda i: (i, 0)),
            pl.BlockSpec((RB, 1), lambda i: (i, 0)),
            pl.BlockSpec((1, D), lambda i: (0, 0)),
            pl.BlockSpec((D, D), lambda i: (0, 0)),
        ],
        out_specs=pl.BlockSpec((RB, D), lambda i: (i, 0)),
        out_shape=jax.ShapeDtypeStruct((N, D), jnp.float32),
    )


def _tc3_body(pa_ref, pb_ref, y2_ref, dinv_ref, b2_ref, bt_ref,
              out_ref, cnt_ref):
    i = pl.program_id(0)
    s = pa_ref[...] + pb_ref[...] + y2_ref[...]
    z = jnp.maximum(dinv_ref[...] * s + b2_ref[...], 0.0)
    bt = bt_ref[0]
    oh = (bt == lax.broadcasted_iota(jnp.int32, (1, B), 1)
          ).astype(jnp.float32)
    blk_sums = lax.dot_general(oh, z, (((0,), (0,)), ((), ())),
                               preferred_element_type=jnp.float32)
    blk_cnt = lax.dot_general(oh, jnp.ones((RB, 1), jnp.float32),
                              (((0,), (0,)), ((), ())),
                              preferred_element_type=jnp.float32)

    @pl.when(i == 0)
    def _():
        out_ref[...] = blk_sums
        cnt_ref[...] = blk_cnt

    @pl.when(i > 0)
    def _():
        out_ref[...] = out_ref[...] + blk_sums
        cnt_ref[...] = cnt_ref[...] + blk_cnt

    @pl.when(i == NB - 1)
    def _():
        out_ref[...] = out_ref[...] / jnp.maximum(cnt_ref[...], 1.0)


@functools.cache
def _tc3_call():
    return pl.pallas_call(
        _tc3_body,
        grid=(NB,),
        in_specs=[
            pl.BlockSpec((RB, D), lambda i: (i, 0)),
            pl.BlockSpec((RB, D), lambda i: (i + NB, 0)),
            pl.BlockSpec((RB, D), lambda i: (i, 0)),
            pl.BlockSpec((RB, 1), lambda i: (i, 0)),
            pl.BlockSpec((1, D), lambda i: (0, 0)),
            pl.BlockSpec((1, RB, 1), lambda i: (i, 0, 0)),
        ],
        out_specs=[
            pl.BlockSpec((B, B), lambda i: (0, 0)),
            pl.BlockSpec((B, 1), lambda i: (0, 0)),
        ],
        out_shape=[
            jax.ShapeDtypeStruct((B, B), jnp.float32),
            jax.ShapeDtypeStruct((B, 1), jnp.float32),
        ],
    )


def kernel(x, edge_index, edge_weight, batch, W1, b1, W2, b2):
    row = edge_index[0].astype(jnp.int32)
    col = edge_index[1].astype(jnp.int32)
    ew = edge_weight

    col2 = col.reshape(NW, EPW)
    ew2 = ew.reshape(NW, EPW)
    row2 = row.reshape(NW, EPW)
    rowm = row2[:, :NFULL * CH].reshape(NW, NFULL, CH)
    rowt = row2[:, NFULL * CH:]
    colm = col2[:, :NFULL * CH].reshape(NW, NFULL, CH)
    colt = col2[:, NFULL * CH:]

    zn = jnp.zeros((N,), jnp.float32)
    z2 = jnp.zeros((N, D), jnp.float32)

    deg_parts = _deg_call()(col2, ew2, zn)
    degT = deg_parts.T
    y1, dinv = _tc1_call()(degT, x, W1)

    parts1 = _msg_call()(y1, rowm, colm, rowt, colt, ew2, z2)
    y2 = _tc2_call()(parts1, parts1, y1, dinv, b1.reshape(1, D), W2)

    parts2 = _msg_call()(y2, rowm, colm, rowt, colt, ew2, z2)
    mean, _ = _tc3_call()(parts2, parts2, y2, dinv, b2.reshape(1, D),
                          batch.astype(jnp.int32).reshape(NB, RB, 1))
    return mean


# speedup vs baseline: 1.1897x; 1.0313x over previous
import functools

import jax
import jax.numpy as jnp
from jax import lax
from jax.experimental import pallas as pl
from jax.experimental.pallas import tpu as pltpu
from jax.experimental.pallas import tpu_sc as plsc

N = 10000
E = 320000
B = 128
D = 128

NC = 2
NS = 16
NW = NC * NS
EPW = E // NW
CH = 40
NFULL = EPW // CH
TAIL = EPW - NFULL * CH
NBUF = 5
RPW = N // NS

NB = 10
RB = N // NB


def _mesh():
    return plsc.VectorSubcoreMesh(core_axis_name="c", subcore_axis_name="s",
                                  num_cores=NC, num_subcores=NS)


def _sc_params():
    return pltpu.CompilerParams(needs_layout_passes=False,
                                use_tc_tiling_on_sc=False)


@functools.cache
def _deg_call():
    @functools.partial(
        pl.kernel,
        out_type=jax.ShapeDtypeStruct((NW, N), jnp.float32),
        mesh=_mesh(),
        compiler_params=_sc_params(),
        scratch_types=[
            pltpu.VMEM((N,), jnp.float32),
            pltpu.VMEM((EPW,), jnp.int32),
            pltpu.VMEM((EPW,), jnp.float32),
        ],
    )
    def deg_kernel(col_hbm, ew_hbm, zn_hbm, out_hbm, deg_v, col_v, ew_v):
        wid = lax.axis_index("s") * NC + lax.axis_index("c")
        pltpu.sync_copy(zn_hbm, deg_v)
        pltpu.sync_copy(col_hbm.at[wid], col_v)
        pltpu.sync_copy(ew_hbm.at[wid], ew_v)

        def body(i, carry):
            idx = col_v[pl.ds(i * 16, 16)]
            w = ew_v[pl.ds(i * 16, 16)]
            plsc.addupdate_scatter(deg_v, [idx], w)
            return carry

        lax.fori_loop(0, EPW // 16, body, 0, unroll=4)
        pltpu.sync_copy(deg_v, out_hbm.at[wid])

    return deg_kernel


@functools.cache
def _msg_call():
    @functools.partial(
        pl.kernel,
        out_type=jax.ShapeDtypeStruct((NC * N, D), jnp.float32),
        mesh=_mesh(),
        compiler_params=_sc_params(),
        scratch_types=[
            pltpu.VMEM_SHARED((N, D), jnp.float32),
            pltpu.VMEM((NFULL, CH), jnp.int32),
            pltpu.VMEM((NFULL, CH), jnp.int32),
            pltpu.VMEM((max(TAIL, 8),), jnp.int32),
            pltpu.VMEM((max(TAIL, 8),), jnp.int32),
            [pltpu.VMEM((CH,), jnp.float32)] * NBUF,
            [pltpu.VMEM((CH, D), jnp.float32)] * NBUF,
            [pltpu.SemaphoreType.DMA] * NBUF,
            [pltpu.SemaphoreType.DMA] * NBUF,
        ],
    )
    def msg_kernel(y_hbm, rowm_hbm, colm_hbm, rowt_hbm, colt_hbm, ew_hbm,
                   z2_hbm, out_hbm,
                   acc_s, rowm_v, colm_v, rowt_v, colt_v, ewc, rows, gsem,
                   ssem):
        cid = lax.axis_index("c")
        sid = lax.axis_index("s")
        wid = sid * NC + cid

        pltpu.sync_copy(rowm_hbm.at[wid], rowm_v)
        pltpu.sync_copy(colm_hbm.at[wid], colm_v)

        def scale_rows(ref, ewc, nrows):
            def srow(i, carry):
                eb = plsc.load_gather(ewc, [jnp.full((16,), i, jnp.int32)])
                for g in range(D // 16):
                    sl = (i, pl.ds(g * 16, 16))
                    ref[sl] = ref[sl] * eb
                return carry
            lax.fori_loop(0, nrows, srow, 0, unroll=8)

        def issue(j, b):
            pltpu.async_copy(y_hbm.at[rowm_v.at[j]], rows[b], gsem[b])
            pltpu.async_copy(ew_hbm.at[wid, pl.ds(j * CH, CH)], ewc[b],
                             gsem[b])

        def drain(b):
            pltpu.make_async_copy(y_hbm.at[pl.ds(0, CH)], rows[b],
                                  gsem[b]).wait()
            pltpu.make_async_copy(ew_hbm.at[0, pl.ds(0, CH)], ewc[b],
                                  gsem[b]).wait()

        def wait_scatter(b):
            pltpu.make_async_copy(y_hbm.at[pl.ds(0, CH)], rows[b],
                                  ssem[b]).wait()

        for jb in range(NBUF - 1):
            issue(jb, jb)

        pltpu.sync_copy(z2_hbm.at[pl.ds(sid * RPW, RPW)],
                        acc_s.at[pl.ds(sid * RPW, RPW)])
        if TAIL:
            pltpu.sync_copy(rowt_hbm.at[wid], rowt_v)
            pltpu.sync_copy(colt_hbm.at[wid], colt_v)
        plsc.subcore_barrier()

        def body(t, carry):
            j0 = NBUF * t
            for b in range(NBUF):
                j = j0 + b
                drain(b)
                scale_rows(rows[b], ewc[b], CH)
                pltpu.async_copy(rows[b], acc_s.at[colm_v.at[j]], ssem[b],
                                 add=True)
                bn = (b + NBUF - 1) % NBUF
                @pl.when(j >= 1)
                def _():
                    wait_scatter(bn)

                @pl.when(j + NBUF - 1 < NFULL)
                def _():
                    issue(j + NBUF - 1, bn)
            return carry

        lax.fori_loop(0, NFULL // NBUF, body, 0)
        wait_scatter((NFULL - 1) % NBUF)

        if TAIL:
            pltpu.async_copy(y_hbm.at[rowt_v], rows[0].at[pl.ds(0, TAIL)],
                             gsem[0]).wait()
            pltpu.sync_copy(ew_hbm.at[wid, pl.ds(NFULL * CH, TAIL)],
                            ewc[0].at[pl.ds(0, TAIL)])
            scale_rows(rows[0], ewc[0], TAIL)
            pltpu.sync_copy(rows[0].at[pl.ds(0, TAIL)], acc_s.at[colt_v],
                            add=True)

        plsc.subcore_barrier()
        pltpu.sync_copy(acc_s.at[pl.ds(sid * RPW, RPW)],
                        out_hbm.at[pl.ds(cid * N + sid * RPW, RPW)])

    return msg_kernel


def _tc1_body(degT_ref, x_ref, w_ref, y_ref, dinv_ref):
    deg = 1.0 + jnp.sum(degT_ref[...], axis=1, keepdims=True)
    dinv = lax.rsqrt(deg)
    xw = jnp.dot(x_ref[...], w_ref[...], preferred_element_type=jnp.float32)
    y_ref[...] = xw * dinv
    dinv_ref[...] = dinv


@functools.cache
def _tc1_call():
    return pl.pallas_call(
        _tc1_body,
        grid=(NB,),
        in_specs=[
            pl.BlockSpec((RB, NW), lambda i: (i, 0)),
            pl.BlockSpec((RB, D), lambda i: (i, 0)),
            pl.BlockSpec((D, D), lambda i: (0, 0)),
        ],
        out_specs=[
            pl.BlockSpec((RB, D), lambda i: (i, 0)),
            pl.BlockSpec((RB, 1), lambda i: (i, 0)),
        ],
        out_shape=[
            jax.ShapeDtypeStruct((N, D), jnp.float32),
            jax.ShapeDtypeStruct((N, 1), jnp.float32),
        ],
    )


def _tc2_body(pa_ref, pb_ref, y1_ref, dinv_ref, b1_ref, w2_ref, y2_ref):
    dinv = dinv_ref[...]
    s = pa_ref[...] + pb_ref[...] + y1_ref[...]
    h = jnp.maximum(dinv * s + b1_ref[...], 0.0)
    y2_ref[...] = jnp.dot(h, w2_ref[...],
                          preferred_element_type=jnp.float32) * dinv


@functools.cache
def _tc2_call():
    return pl.pallas_call(
        _tc2_body,
        grid=(NB,),
        in_specs=[
            pl.BlockSpec((RB, D), lambda i: (i, 0)),
            pl.BlockSpec((RB, D), lambda i: (i + NB, 0)),
            pl.BlockSpec((RB, D), lambda i: (i, 0)),
            pl.BlockSpec((RB, 1), lambda i: (i, 0)),
            pl.BlockSpec((1, D), lambda i: (0, 0)),
            pl.BlockSpec((D, D), lambda i: (0, 0)),
        ],
        out_specs=pl.BlockSpec((RB, D), lambda i: (i, 0)),
        out_shape=jax.ShapeDtypeStruct((N, D), jnp.float32),
    )


def _tc3_body(pa_ref, pb_ref, y2_ref, dinv_ref, b2_ref, bt_ref,
              out_ref, cnt_ref):
    i = pl.program_id(0)
    s = pa_ref[...] + pb_ref[...] + y2_ref[...]
    z = jnp.maximum(dinv_ref[...] * s + b2_ref[...], 0.0)
    bt = bt_ref[0]
    oh = (bt == lax.broadcasted_iota(jnp.int32, (1, B), 1)
          ).astype(jnp.float32)
    blk_sums = lax.dot_general(oh, z, (((0,), (0,)), ((), ())),
                               preferred_element_type=jnp.float32)
    blk_cnt = lax.dot_general(oh, jnp.ones((RB, 1), jnp.float32),
                              (((0,), (0,)), ((), ())),
                              preferred_element_type=jnp.float32)

    @pl.when(i == 0)
    def _():
        out_ref[...] = blk_sums
        cnt_ref[...] = blk_cnt

    @pl.when(i > 0)
    def _():
        out_ref[...] = out_ref[...] + blk_sums
        cnt_ref[...] = cnt_ref[...] + blk_cnt

    @pl.when(i == NB - 1)
    def _():
        out_ref[...] = out_ref[...] / jnp.maximum(cnt_ref[...], 1.0)


@functools.cache
def _tc3_call():
    return pl.pallas_call(
        _tc3_body,
        grid=(NB,),
        in_specs=[
            pl.BlockSpec((RB, D), lambda i: (i, 0)),
            pl.BlockSpec((RB, D), lambda i: (i + NB, 0)),
            pl.BlockSpec((RB, D), lambda i: (i, 0)),
            pl.BlockSpec((RB, 1), lambda i: (i, 0)),
            pl.BlockSpec((1, D), lambda i: (0, 0)),
            pl.BlockSpec((1, RB, 1), lambda i: (i, 0, 0)),
        ],
        out_specs=[
            pl.BlockSpec((B, B), lambda i: (0, 0)),
            pl.BlockSpec((B, 1), lambda i: (0, 0)),
        ],
        out_shape=[
            jax.ShapeDtypeStruct((B, B), jnp.float32),
            jax.ShapeDtypeStruct((B, 1), jnp.float32),
        ],
    )


def kernel(x, edge_index, edge_weight, batch, W1, b1, W2, b2):
    row = edge_index[0].astype(jnp.int32)
    col = edge_index[1].astype(jnp.int32)
    ew = edge_weight

    col2 = col.reshape(NW, EPW)
    ew2 = ew.reshape(NW, EPW)
    row2 = row.reshape(NW, EPW)
    rowm = row2[:, :NFULL * CH].reshape(NW, NFULL, CH)
    colm = col2[:, :NFULL * CH].reshape(NW, NFULL, CH)
    if TAIL:
        rowt = row2[:, NFULL * CH:]
        colt = col2[:, NFULL * CH:]
    else:
        rowt = row2[:, :8]
        colt = col2[:, :8]

    zn = jnp.zeros((N,), jnp.float32)
    z2 = jnp.zeros((N, D), jnp.float32)

    deg_parts = _deg_call()(col2, ew2, zn)
    degT = deg_parts.T
    y1, dinv = _tc1_call()(degT, x, W1)

    parts1 = _msg_call()(y1, rowm, colm, rowt, colt, ew2, z2)
    y2 = _tc2_call()(parts1, parts1, y1, dinv, b1.reshape(1, D), W2)

    parts2 = _msg_call()(y2, rowm, colm, rowt, colt, ew2, z2)
    mean, _ = _tc3_call()(parts2, parts2, y2, dinv, b2.reshape(1, D),
                          batch.astype(jnp.int32).reshape(NB, RB, 1))
    return mean
